# baseline (speedup 1.0000x reference)
import functools

import jax
import jax.numpy as jnp
from jax import lax
from jax.experimental import pallas as pl
from jax.experimental.pallas import tpu as pltpu
from jax.experimental.pallas import tpu_sc as plsc

NUM_VARS = 26
VOCAB = 100002
DIM = 32
B = 16384
L = 20
NW = 32
BROWS_W = B // NW
CB = 64
CW = CB * L
NCHUNK = BROWS_W // CB
LANES = 16
R = 2
CB_SHIFT = CB.bit_length() - 1

_mesh = plsc.VectorSubcoreMesh(core_axis_name="c", subcore_axis_name="s")


@functools.partial(
    pl.kernel,
    mesh=_mesh,
    out_type=jax.ShapeDtypeStruct((B, L, NUM_VARS * DIM), jnp.float32),
    scratch_types=[
        pltpu.VMEM((R, CB, L), jnp.int32),
        pltpu.VMEM((R, 1, CW), jnp.int32),
        pltpu.VMEM((R, CW, DIM), jnp.float32),
        pltpu.SemaphoreType.DMA((R,)),
        pltpu.SemaphoreType.DMA((R,)),
    ],
    compiler_params=pltpu.CompilerParams(use_tc_tiling_on_sc=False,
                                         needs_layout_passes=False),
)
def _bank(*refs):
    inputs = refs[:NUM_VARS]
    tables = refs[NUM_VARS:2 * NUM_VARS]
    out = refs[2 * NUM_VARS]
    idx_v, idx_t, rows_v, gsem, ssem = refs[2 * NUM_VARS + 1:]

    wid = lax.axis_index("s") * 2 + lax.axis_index("c")
    wbase = wid * BROWS_W

    def chunk_body(ci, _):
        rbase = wbase + ci * CB

        def load_clamp_gather(s):
            r = s % R
            pltpu.sync_copy(inputs[s].at[pl.ds(rbase, CB), :], idx_v.at[r])

            def clamp_t_body(j, _):
                jf = j * LANES + lax.iota(jnp.int32, LANES)
                bv = jnp.bitwise_and(jf, CB - 1)
                lv = jnp.right_shift(jf, CB_SHIFT)
                v = plsc.load_gather(idx_v.at[r], [bv, lv])
                v = jnp.where(v == -1, VOCAB - 1, v)
                v = jnp.minimum(jnp.maximum(v, 0), VOCAB - 1)
                idx_t[r, 0, pl.ds(j * LANES, LANES)] = v
                return _

            lax.fori_loop(0, CW // LANES, clamp_t_body, None)
            pltpu.async_copy(tables[s].at[idx_t.at[r, 0]], rows_v.at[r],
                             gsem.at[r])

        def scatter(s):
            r = s % R
            pltpu.make_async_copy(tables[s].at[idx_t.at[r, 0]], rows_v.at[r],
                                  gsem.at[r]).wait()

            def scat_body(l, _):
                pltpu.async_copy(rows_v.at[r, pl.ds(l * CB, CB)],
                                 out.at[pl.ds(rbase, CB), l,
                                        pl.ds(s * DIM, DIM)],
                                 ssem.at[r])
                return _

            lax.fori_loop(0, L, scat_body, None)

        def drain_scatter(s):
            r = s % R

            def drain_body(l, _):
                pltpu.make_async_copy(rows_v.at[r, pl.ds(l * CB, CB)],
                                      out.at[pl.ds(rbase, CB), l,
                                             pl.ds(s * DIM, DIM)],
                                      ssem.at[r]).wait()
                return _

            lax.fori_loop(0, L, drain_body, None)

        for s in range(NUM_VARS):
            if s >= R:
                drain_scatter(s - R)
            load_clamp_gather(s)
            if s >= 1:
                scatter(s - 1)
        scatter(NUM_VARS - 1)
        drain_scatter(NUM_VARS - 2)
        drain_scatter(NUM_VARS - 1)
        return _

    lax.fori_loop(0, NCHUNK, chunk_body, None)


def kernel(inputs_0, inputs_1, inputs_2, inputs_3, inputs_4, inputs_5, inputs_6, inputs_7, inputs_8, inputs_9, inputs_10, inputs_11, inputs_12, inputs_13, inputs_14, inputs_15, inputs_16, inputs_17, inputs_18, inputs_19, inputs_20, inputs_21, inputs_22, inputs_23, inputs_24, inputs_25, table_0, table_1, table_2, table_3, table_4, table_5, table_6, table_7, table_8, table_9, table_10, table_11, table_12, table_13, table_14, table_15, table_16, table_17, table_18, table_19, table_20, table_21, table_22, table_23, table_24, table_25):
    args = locals()
    ins = [args[f"inputs_{i}"] for i in range(NUM_VARS)]
    tabs = [args[f"table_{i}"] for i in range(NUM_VARS)]
    return _bank(*ins, *tabs)
